# baseline (speedup 1.0000x reference)
import functools

import jax
import jax.numpy as jnp
from jax import lax
from jax.experimental import pallas as pl
from jax.experimental.pallas import tpu as pltpu
from jax.experimental.pallas import tpu_sc as plsc

_LANES = 16
_NC, _NS = 2, 16
_NW = _NC * _NS

_T = 256
_BU = 16


@functools.cache
def _build_table_kernel(K, B, D):
    K2 = K // 2
    D2 = 2 * D
    n_pairs = K2 * B
    blocks_per_w = n_pairs // _NW
    assert blocks_per_w % 2 == 0

    mesh = plsc.VectorSubcoreMesh(core_axis_name="c", subcore_axis_name="s")

    @functools.partial(
        pl.kernel,
        mesh=mesh,
        out_type=jax.ShapeDtypeStruct((K2 * B * B, D2), jnp.float32),
        compiler_params=pltpu.CompilerParams(use_tc_tiling_on_sc=False),
        scratch_types=[
            pltpu.VMEM((blocks_per_w * D + _LANES,), jnp.float32),
            pltpu.VMEM((D + B * D + _LANES,), jnp.float32),
            pltpu.VMEM((B, D2), jnp.float32),
            pltpu.VMEM((B, D2), jnp.float32),
            pltpu.SemaphoreType.DMA,
            pltpu.SemaphoreType.DMA,
        ],
    )
    def build(cb_hbm, tab_hbm, left_v, right_v, blk_a, blk_b, sem_a, sem_b):
        wid = lax.axis_index("s") * _NC + lax.axis_index("c")
        lane = lax.iota(jnp.int32, _LANES)
        left_half = lane < D
        j2 = wid // (B // blocks_per_w)
        hi0 = (wid % (B // blocks_per_w)) * blocks_per_w
        blk = (blk_a, blk_b)
        sem = (sem_a, sem_b)

        pltpu.sync_copy(
            cb_hbm.at[pl.ds((2 * j2 * B + hi0) * D, blocks_per_w * D)],
            left_v.at[pl.ds(0, blocks_per_w * D)])
        pltpu.sync_copy(
            cb_hbm.at[pl.ds((2 * j2 + 1) * B * D, B * D)],
            right_v.at[pl.ds(D, B * D)])

        def out_copy(h, p):
            pid = (wid * blocks_per_w + h) * B
            return pltpu.make_async_copy(
                blk[p], tab_hbm.at[pl.ds(pid, B)], sem[p])

        def fill(h, p):
            lv = left_v[pl.ds(h * D, _LANES)]

            def rows(r, carry):
                for u in range(_BU):
                    lo = r * _BU + u
                    rv = right_v[pl.ds(lo * D, _LANES)]
                    blk[p][lo, :] = jnp.where(left_half, lv, rv)
                return carry

            lax.fori_loop(0, B // _BU, rows, 0)

        def loop(h2, carry):
            for p in (0, 1):
                h = h2 * 2 + p
                @pl.when(h2 >= 1)
                def _():
                    out_copy(h - 2, p).wait()
                fill(h, p)
                out_copy(h, p).start()
            return carry

        lax.fori_loop(0, blocks_per_w // 2, loop, 0)
        out_copy(blocks_per_w - 2, 0).wait()
        out_copy(blocks_per_w - 1, 1).wait()

    return build


@functools.cache
def _build_main_kernel(N, K, B, D):
    K2 = K // 2
    D2 = 2 * D
    rows_per_w = N // _NW
    steps = rows_per_w // _T
    idx_per_step = _T * K2
    assert steps % 2 == 0 and steps >= 4

    mesh = plsc.VectorSubcoreMesh(core_axis_name="c", subcore_axis_name="s")

    @functools.partial(
        pl.kernel,
        mesh=mesh,
        out_type=jax.ShapeDtypeStruct((N * K2, D2), jnp.float32),
        compiler_params=pltpu.CompilerParams(use_tc_tiling_on_sc=False),
        scratch_types=[
            pltpu.VMEM((_T * K // 128, 128), jnp.int32),
            pltpu.VMEM((_T * K // 128, 128), jnp.int32),
            pltpu.VMEM((idx_per_step,), jnp.int32),
            pltpu.VMEM((idx_per_step,), jnp.int32),
            pltpu.VMEM((idx_per_step, D2), jnp.float32),
            pltpu.VMEM((idx_per_step, D2), jnp.float32),
            pltpu.SemaphoreType.DMA,
            pltpu.SemaphoreType.DMA,
            pltpu.SemaphoreType.DMA,
            pltpu.SemaphoreType.DMA,
            pltpu.SemaphoreType.DMA,
            pltpu.SemaphoreType.DMA,
        ],
    )
    def pq(idx_hbm, pair_hbm, out_hbm,
           idx2_a, idx2_b, idx1_a, idx1_b, rows_a, rows_b,
           semi_a, semi_b, semg_a, semg_b, sems_a, sems_b):
        wid = lax.axis_index("s") * _NC + lax.axis_index("c")
        lane = lax.iota(jnp.int32, _LANES)
        j2 = lane % K2
        j2_base = j2 * (B * B)
        low_half = lane < K2
        perm_e = (lane % K2) * 2
        perm_o = perm_e + 1
        idx2 = (idx2_a, idx2_b)
        idx1 = (idx1_a, idx1_b)
        rows = (rows_a, rows_b)
        semi = (semi_a, semi_b)
        semg = (semg_a, semg_b)
        sems = (sems_a, sems_b)

        def idx_copy(t, p):
            row = (wid * rows_per_w + t * _T) * K // 128
            return pltpu.make_async_copy(
                idx_hbm.at[pl.ds(row, _T * K // 128), :], idx2[p], semi[p])

        def gather_copy(t, p):
            return pltpu.make_async_copy(pair_hbm.at[idx1[p]], rows[p], semg[p])

        def store_copy(t, p):
            base = (wid * rows_per_w + t * _T) * K2
            return pltpu.make_async_copy(
                rows[p], out_hbm.at[pl.ds(base, idx_per_step)], sems[p])

        def permute(vec, perm):
            dnums = lax.GatherDimensionNumbers(
                offset_dims=(), collapsed_slice_dims=(0,), start_index_map=(0,))
            return lax.gather(vec, perm[:, None], dnums, slice_sizes=(1,),
                              mode=lax.GatherScatterMode.PROMISE_IN_BOUNDS)

        def make_pair_idx(p):
            def body(c, carry):
                va = 2 * c
                row_a = idx2[p][va >> 3, pl.ds((va & 7) * K, K)]
                row_b = idx2[p][(va + 1) >> 3, pl.ds(((va + 1) & 7) * K, K)]
                hi = jnp.where(low_half, permute(row_a, perm_e),
                               permute(row_b, perm_e))
                lo = jnp.where(low_half, permute(row_a, perm_o),
                               permute(row_b, perm_o))
                idx1[p][pl.ds(c * _LANES, _LANES)] = hi * B + lo + j2_base
                return carry
            lax.fori_loop(0, idx_per_step // _LANES, body, 0)

        idx_copy(0, 0).start()
        idx_copy(1, 1).start()
        idx_copy(0, 0).wait()
        make_pair_idx(0)

        def loop(t2, carry):
            for p in (0, 1):
                t = t2 * 2 + p
                @pl.when(t2 >= 1)
                def _():
                    store_copy(t - 2, p).wait()
                gather_copy(t, p).start()
                @pl.when(t + 2 < steps)
                def _():
                    idx_copy(t + 2, p).start()
                @pl.when(t + 1 < steps)
                def _():
                    idx_copy(t + 1, 1 - p).wait()
                    make_pair_idx(1 - p)
                gather_copy(t, p).wait()
                store_copy(t, p).start()
            return carry

        lax.fori_loop(0, steps // 2, loop, 0)
        store_copy(steps - 2, 0).wait()
        store_copy(steps - 1, 1).wait()

    return pq


def kernel(indices, codebook):
    N, K = indices.shape
    _, B, D = codebook.shape
    pair = _build_table_kernel(K, B, D)(codebook.reshape(-1))
    out = _build_main_kernel(N, K, B, D)(indices.reshape(N * K // 128, 128), pair)
    return out.reshape(N, K * D)


# speedup vs baseline: 1.0741x; 1.0741x over previous
import functools

import jax
import jax.numpy as jnp
from jax import lax
from jax.experimental import pallas as pl
from jax.experimental.pallas import tpu as pltpu
from jax.experimental.pallas import tpu_sc as plsc

_LANES = 16
_NC, _NS = 2, 16
_NW = _NC * _NS

_T = 256
_BU = 16


@functools.cache
def _build_table_kernel(K, B, D):
    K2 = K // 2
    D2 = 2 * D
    n_pairs = K2 * B
    blocks_per_w = n_pairs // _NW
    assert blocks_per_w % 2 == 0

    mesh = plsc.VectorSubcoreMesh(core_axis_name="c", subcore_axis_name="s")

    @functools.partial(
        pl.kernel,
        mesh=mesh,
        out_type=jax.ShapeDtypeStruct((K2 * B * B, D2), jnp.float32),
        compiler_params=pltpu.CompilerParams(use_tc_tiling_on_sc=False),
        scratch_types=[
            pltpu.VMEM((blocks_per_w * D + _LANES,), jnp.float32),
            pltpu.VMEM((D + B * D + _LANES,), jnp.float32),
            pltpu.VMEM((B, D2), jnp.float32),
            pltpu.VMEM((B, D2), jnp.float32),
            pltpu.SemaphoreType.DMA,
            pltpu.SemaphoreType.DMA,
        ],
    )
    def build(cb_hbm, tab_hbm, left_v, right_v, blk_a, blk_b, sem_a, sem_b):
        wid = lax.axis_index("s") * _NC + lax.axis_index("c")
        lane = lax.iota(jnp.int32, _LANES)
        left_half = lane < D
        j2 = wid // (B // blocks_per_w)
        hi0 = (wid % (B // blocks_per_w)) * blocks_per_w
        blk = (blk_a, blk_b)
        sem = (sem_a, sem_b)

        pltpu.sync_copy(
            cb_hbm.at[pl.ds((2 * j2 * B + hi0) * D, blocks_per_w * D)],
            left_v.at[pl.ds(0, blocks_per_w * D)])
        pltpu.sync_copy(
            cb_hbm.at[pl.ds((2 * j2 + 1) * B * D, B * D)],
            right_v.at[pl.ds(D, B * D)])

        def out_copy(h, p):
            pid = (wid * blocks_per_w + h) * B
            return pltpu.make_async_copy(
                blk[p], tab_hbm.at[pl.ds(pid, B)], sem[p])

        def fill(h, p):
            lv = left_v[pl.ds(h * D, _LANES)]

            def rows(r, carry):
                for u in range(_BU):
                    lo = r * _BU + u
                    rv = right_v[pl.ds(lo * D, _LANES)]
                    blk[p][lo, :] = jnp.where(left_half, lv, rv)
                return carry

            lax.fori_loop(0, B // _BU, rows, 0)

        def loop(h2, carry):
            for p in (0, 1):
                h = h2 * 2 + p
                @pl.when(h2 >= 1)
                def _():
                    out_copy(h - 2, p).wait()
                fill(h, p)
                out_copy(h, p).start()
            return carry

        lax.fori_loop(0, blocks_per_w // 2, loop, 0)
        out_copy(blocks_per_w - 2, 0).wait()
        out_copy(blocks_per_w - 1, 1).wait()

    return build


@functools.cache
def _build_main_kernel(N, K, B, D):
    K2 = K // 2
    D2 = 2 * D
    rows_per_w = N // _NW
    steps = rows_per_w // _T
    idx_per_step = _T * K2
    assert steps % 2 == 0 and steps >= 4

    mesh = plsc.VectorSubcoreMesh(core_axis_name="c", subcore_axis_name="s")

    @functools.partial(
        pl.kernel,
        mesh=mesh,
        out_type=jax.ShapeDtypeStruct((N * K2, D2), jnp.float32),
        compiler_params=pltpu.CompilerParams(use_tc_tiling_on_sc=False),
        scratch_types=[
            pltpu.VMEM((K * _T,), jnp.int32),
            pltpu.VMEM((K * _T,), jnp.int32),
            pltpu.VMEM((idx_per_step,), jnp.int32),
            pltpu.VMEM((idx_per_step,), jnp.int32),
            pltpu.VMEM((idx_per_step, D2), jnp.float32),
            pltpu.VMEM((idx_per_step, D2), jnp.float32),
            pltpu.SemaphoreType.DMA,
            pltpu.SemaphoreType.DMA,
            pltpu.SemaphoreType.DMA,
            pltpu.SemaphoreType.DMA,
            pltpu.SemaphoreType.DMA,
            pltpu.SemaphoreType.DMA,
        ],
    )
    def pq(idx_hbm, pair_hbm, out_hbm,
           idx2_a, idx2_b, idx1_a, idx1_b, rows_a, rows_b,
           semi_a, semi_b, semg_a, semg_b, sems_a, sems_b):
        wid = lax.axis_index("s") * _NC + lax.axis_index("c")
        lane = lax.iota(jnp.int32, _LANES)
        j2 = lane % K2
        j2_base = j2 * (B * B)
        low_half = lane < K2
        perm_e = (lane % K2) * 2
        perm_o = perm_e + 1
        idx2 = (idx2_a, idx2_b)
        idx1 = (idx1_a, idx1_b)
        rows = (rows_a, rows_b)
        semi = (semi_a, semi_b)
        semg = (semg_a, semg_b)
        sems = (sems_a, sems_b)

        def idx_copies(t, p):
            row = wid * rows_per_w + t * _T
            return [pltpu.make_async_copy(
                        idx_hbm.at[pl.ds(j * N + row, _T)],
                        idx2[p].at[pl.ds(j * _T, _T)], semi[p])
                    for j in range(K)]

        def idx_start(t, p):
            for cp in idx_copies(t, p):
                cp.start()

        def idx_wait(t, p):
            for cp in idx_copies(t, p):
                cp.wait()

        def gather_copy(t, p):
            return pltpu.make_async_copy(pair_hbm.at[idx1[p]], rows[p], semg[p])

        def store_copy(t, p):
            base = (wid * rows_per_w + t * _T) * K2
            return pltpu.make_async_copy(
                rows[p], out_hbm.at[pl.ds(base, idx_per_step)], sems[p])

        def permute(vec, perm):
            dnums = lax.GatherDimensionNumbers(
                offset_dims=(), collapsed_slice_dims=(0,), start_index_map=(0,))
            return lax.gather(vec, perm[:, None], dnums, slice_sizes=(1,),
                              mode=lax.GatherScatterMode.PROMISE_IN_BOUNDS)

        xor_perms = {s: lane ^ s for s in (1, 2, 4, 8)}

        def make_pair_idx(p):
            def body(c, carry):
                m = [idx2[p][pl.ds(j * _T + c * _LANES, _LANES)]
                     for j in range(K)]
                for s in (1, 2, 4, 8):
                    bit = (lane & s) == 0
                    nm = list(m)
                    for i in range(K):
                        if i & s:
                            continue
                        a, b = m[i], m[i | s]
                        nm[i] = jnp.where(bit, a, permute(b, xor_perms[s]))
                        nm[i | s] = jnp.where(bit, permute(a, xor_perms[s]), b)
                    m = nm
                for u in range(_LANES // 2):
                    hi = jnp.where(low_half, permute(m[2 * u], perm_e),
                                   permute(m[2 * u + 1], perm_e))
                    lo = jnp.where(low_half, permute(m[2 * u], perm_o),
                                   permute(m[2 * u + 1], perm_o))
                    q = (c * _LANES + 2 * u) * K2
                    idx1[p][pl.ds(q, _LANES)] = hi * B + lo + j2_base
                return carry
            lax.fori_loop(0, _T // _LANES, body, 0)

        idx_start(0, 0)
        idx_start(1, 1)
        idx_wait(0, 0)
        make_pair_idx(0)

        def loop(t2, carry):
            for p in (0, 1):
                t = t2 * 2 + p
                @pl.when(t2 >= 1)
                def _():
                    store_copy(t - 2, p).wait()
                gather_copy(t, p).start()
                @pl.when(t + 2 < steps)
                def _():
                    idx_start(t + 2, p)
                @pl.when(t + 1 < steps)
                def _():
                    idx_wait(t + 1, 1 - p)
                    make_pair_idx(1 - p)
                gather_copy(t, p).wait()
                store_copy(t, p).start()
            return carry

        lax.fori_loop(0, steps // 2, loop, 0)
        store_copy(steps - 2, 0).wait()
        store_copy(steps - 1, 1).wait()

    return pq


def kernel(indices, codebook):
    N, K = indices.shape
    _, B, D = codebook.shape
    pair = _build_table_kernel(K, B, D)(codebook.reshape(-1))
    out = _build_main_kernel(N, K, B, D)(indices.T.reshape(-1), pair)
    return out.reshape(N, K * D)
